# baseline (speedup 1.0000x reference)
import functools

import jax
import jax.numpy as jnp
from jax import lax
from jax.experimental import pallas as pl
from jax.experimental.pallas import tpu as pltpu
from jax.experimental.pallas import tpu_sc as plsc

N_CLUSTERS = 4
HID = 128
N_GRAPHS = 8

NP = 10240
NW = 32
CHUNK = 128
NCHUNK = 80
NBUF = 2
HCHUNK = 40
EP = NW * NCHUNK * CHUNK
ROWS_PER_TILE = NP // 16


def _deg_body(col_hbm, ones_hbm, zeros_hbm, out_hbm, colbuf, onesbuf, acc,
              dsem):
    cid = lax.axis_index("c")
    sid = lax.axis_index("s")
    w = cid * 16 + sid
    base = sid * ROWS_PER_TILE
    pltpu.sync_copy(zeros_hbm.at[pl.ds(base, ROWS_PER_TILE)],
                    acc.at[pl.ds(base, ROWS_PER_TILE)])
    pltpu.sync_copy(col_hbm.at[w], colbuf)
    pltpu.sync_copy(ones_hbm, onesbuf)
    plsc.subcore_barrier()

    def body(j, carry):
        pltpu.async_copy(onesbuf, acc.at[colbuf.at[j]], dsem, add=True)

        @pl.when(j >= 8)
        def _():
            pltpu.make_async_copy(onesbuf, acc.at[colbuf.at[0]], dsem).wait()

        return carry

    lax.fori_loop(0, NCHUNK, body, 0)

    def drain(j, carry):
        pltpu.make_async_copy(onesbuf, acc.at[colbuf.at[0]], dsem).wait()
        return carry

    lax.fori_loop(0, 8, drain, 0)
    plsc.subcore_barrier()
    pltpu.sync_copy(acc.at[pl.ds(base, ROWS_PER_TILE)],
                    out_hbm.at[cid, pl.ds(base, ROWS_PER_TILE)])


def _seg_body(tab_hbm, idx_hbm, col_hbm, zeros_hbm, out_hbm,
              idxbuf, colbuf, rows_v, acc, gsem0, gsem1):
    cid = lax.axis_index("c")
    sid = lax.axis_index("s")
    w = cid * 16 + sid
    base = sid * ROWS_PER_TILE
    pltpu.sync_copy(zeros_hbm.at[pl.ds(base, ROWS_PER_TILE)],
                    acc.at[pl.ds(base, ROWS_PER_TILE)])
    plsc.subcore_barrier()

    gsems = (gsem0, gsem1)
    for ph in range(2):
        pltpu.sync_copy(idx_hbm.at[w, pl.ds(ph * HCHUNK, HCHUNK)], idxbuf)
        pltpu.sync_copy(col_hbm.at[w, pl.ds(ph * HCHUNK, HCHUNK)], colbuf)
        for b in range(NBUF):
            pltpu.async_copy(tab_hbm.at[idxbuf.at[b]], rows_v.at[b], gsems[b])

        def group(g, carry):
            for b in range(NBUF):
                j = g * NBUF + b
                pltpu.make_async_copy(tab_hbm.at[idxbuf.at[b]], rows_v.at[b],
                                      gsems[b]).wait()
                pltpu.sync_copy(rows_v.at[b], acc.at[colbuf.at[j]], add=True)

                @pl.when(j + NBUF < HCHUNK)
                def _():
                    pltpu.async_copy(tab_hbm.at[idxbuf.at[j + NBUF]],
                                     rows_v.at[b], gsems[b])

            return carry

        lax.fori_loop(0, HCHUNK // NBUF, group, 0)

    plsc.subcore_barrier()
    pltpu.sync_copy(acc.at[pl.ds(base, ROWS_PER_TILE)],
                    out_hbm.at[cid, pl.ds(base, ROWS_PER_TILE)])


@functools.lru_cache(maxsize=None)
def _sc_kernels():
    mesh = plsc.VectorSubcoreMesh(core_axis_name="c", subcore_axis_name="s")
    deg = pl.kernel(
        _deg_body,
        out_type=jax.ShapeDtypeStruct((2, NP, HID), jnp.float32),
        mesh=mesh,
        scratch_types=[
            pltpu.VMEM((NCHUNK, CHUNK), jnp.int32),
            pltpu.VMEM((CHUNK, HID), jnp.float32),
            pltpu.VMEM_SHARED((NP, HID), jnp.float32),
            pltpu.SemaphoreType.DMA,
        ],
    )
    seg = pl.kernel(
        _seg_body,
        out_type=jax.ShapeDtypeStruct((2, NP, HID), jnp.float32),
        mesh=mesh,
        scratch_types=[
            pltpu.VMEM((HCHUNK, CHUNK), jnp.int32),
            pltpu.VMEM((HCHUNK, CHUNK), jnp.int32),
            pltpu.VMEM((NBUF, CHUNK, HID), jnp.float32),
            pltpu.VMEM_SHARED((NP, HID), jnp.float32),
            pltpu.SemaphoreType.DMA,
            pltpu.SemaphoreType.DMA,
        ],
    )
    return deg, seg


_BLK = 1024


def _dinv_body(degp_ref, dinv_ref):
    deg = degp_ref[0, :, 0] + degp_ref[1, :, 0] + 1.0
    dinv_ref[...] = lax.rsqrt(deg)[:, None]


def _dinv(degp):
    grid = NP // _BLK
    return pl.pallas_call(
        _dinv_body,
        grid=(grid,),
        in_specs=[pl.BlockSpec((2, _BLK, HID), lambda i: (0, i, 0))],
        out_specs=pl.BlockSpec((_BLK, 1), lambda i: (i, 0)),
        out_shape=jax.ShapeDtypeStruct((NP, 1), jnp.float32),
    )(degp)


def _ys_body(h_ref, dinv_ref, w_ref, ys_ref):
    h = h_ref[...]
    dinv = dinv_ref[...]
    for c in range(N_CLUSTERS):
        ys_ref[c] = dinv * jnp.dot(h, w_ref[c],
                                   preferred_element_type=jnp.float32)


def _ys(h, dinv, w4):
    grid = NP // _BLK
    return pl.pallas_call(
        _ys_body,
        grid=(grid,),
        in_specs=[
            pl.BlockSpec((_BLK, HID), lambda i: (i, 0)),
            pl.BlockSpec((_BLK, 1), lambda i: (i, 0)),
            pl.BlockSpec((N_CLUSTERS, HID, HID), lambda i: (0, 0, 0)),
        ],
        out_specs=pl.BlockSpec((N_CLUSTERS, _BLK, HID), lambda i: (0, i, 0)),
        out_shape=jax.ShapeDtypeStruct((N_CLUSTERS, NP, HID), jnp.float32),
    )(h, dinv, w4)


def _layer_body(emit_x2, seg_ref, ys_ref, dinv_ref, lab_ref, bat_ref,
                *out_refs):
    i = pl.program_id(0)
    lab = lab_ref[...]
    ys_sel = jnp.zeros((_BLK, HID), jnp.float32)
    for c in range(N_CLUSTERS):
        ys_sel = jnp.where(lab == c, ys_ref[c], ys_sel)
    x2 = dinv_ref[...] * (seg_ref[0] + seg_ref[1] + ys_sel)
    x2 = jnp.maximum(x2, 0.0)
    if emit_x2:
        out_refs[0][...] = x2
    pooled_ref = out_refs[-1]
    onehot = (bat_ref[...] == lax.broadcasted_iota(jnp.int32, (1, N_GRAPHS), 1)
              ).astype(jnp.float32)
    p = lax.dot_general(onehot, x2, (((0,), (0,)), ((), ())),
                        preferred_element_type=jnp.float32,
                        precision=lax.Precision.HIGHEST)

    @pl.when(i == 0)
    def _():
        pooled_ref[...] = jnp.zeros_like(pooled_ref)

    pooled_ref[...] += p


def _layer(segp, ys, dinv, lab, batch, emit_x2):
    grid = NP // _BLK
    out_shape = [jax.ShapeDtypeStruct((N_GRAPHS, HID), jnp.float32)]
    out_specs = [pl.BlockSpec((N_GRAPHS, HID), lambda i: (0, 0))]
    if emit_x2:
        out_shape.insert(0, jax.ShapeDtypeStruct((NP, HID), jnp.float32))
        out_specs.insert(0, pl.BlockSpec((_BLK, HID), lambda i: (i, 0)))
    return pl.pallas_call(
        functools.partial(_layer_body, emit_x2),
        grid=(grid,),
        in_specs=[
            pl.BlockSpec((2, _BLK, HID), lambda i: (0, i, 0)),
            pl.BlockSpec((N_CLUSTERS, _BLK, HID), lambda i: (0, i, 0)),
            pl.BlockSpec((_BLK, 1), lambda i: (i, 0)),
            pl.BlockSpec((_BLK, 1), lambda i: (i, 0)),
            pl.BlockSpec((_BLK, 1), lambda i: (i, 0)),
        ],
        out_specs=out_specs,
        out_shape=out_shape,
    )(segp, ys, dinv, lab, batch)


def _head_body(p1_ref, p2_ref, w1_ref, b1_ref, g_ref, b_ref, w2_ref, b2_ref,
               out_ref):
    z = jnp.concatenate([p1_ref[...], p2_ref[...]], axis=1)
    z1 = jnp.dot(z, w1_ref[...], preferred_element_type=jnp.float32) + b1_ref[...]
    mean = jnp.mean(z1, axis=0, keepdims=True)
    var = jnp.mean(jnp.square(z1 - mean), axis=0, keepdims=True)
    zn = (z1 - mean) * lax.rsqrt(var + 1e-5) * g_ref[...] + b_ref[...]
    zn = jnp.maximum(zn, 0.0)
    out_ref[...] = jnp.dot(zn, w2_ref[...],
                           preferred_element_type=jnp.float32) + b2_ref[...]


def _head(p1, p2, w1, b1, gamma, beta, w2, b2):
    return pl.pallas_call(
        _head_body,
        out_shape=jax.ShapeDtypeStruct((N_GRAPHS, 64), jnp.float32),
    )(p1, p2, w1, b1.reshape(1, -1), gamma.reshape(1, -1),
      beta.reshape(1, -1), w2, b2.reshape(1, -1))


def kernel(x, edge_index, batch, conv_w, mlp_w1, mlp_b1, bn_gamma, bn_beta,
           mlp_w2, mlp_b2):
    lab = x[:, :1].astype(jnp.int32)
    h = x[:, 1:]
    n, _ = h.shape

    h_pad = jnp.pad(h, ((0, NP - n), (0, 0)))
    lab_pad = jnp.pad(lab, ((0, NP - n), (0, 0)))
    batch_pad = jnp.pad(batch.astype(jnp.int32), (0, NP - n)).reshape(NP, 1)

    row = edge_index[0].astype(jnp.int32)
    col = edge_index[1].astype(jnp.int32)
    e = row.shape[0]
    rowp = jnp.pad(row, (0, EP - e), constant_values=n)
    colp = jnp.pad(col, (0, EP - e), constant_values=n)
    lab_col = lab_pad[colp, 0]
    sf, sc = lax.sort([lab_col * NP + rowp, colp], num_keys=1)
    fused = sf.reshape(NW, NCHUNK, CHUNK)
    col3 = sc.reshape(NW, NCHUNK, CHUNK)

    ones128 = jnp.ones((CHUNK, HID), jnp.float32)
    zeros128 = jnp.zeros((NP, HID), jnp.float32)

    deg_kernel, seg_kernel = _sc_kernels()
    degp = deg_kernel(col3, ones128, zeros128)
    dinv = _dinv(degp)

    ys1 = _ys(h_pad, dinv, conv_w[:N_CLUSTERS]).reshape(N_CLUSTERS * NP, HID)
    segp1 = seg_kernel(ys1, fused, col3, zeros128)
    x2, pooled1 = _layer(segp1, ys1.reshape(N_CLUSTERS, NP, HID), dinv,
                         lab_pad, batch_pad, emit_x2=True)

    ys2 = _ys(x2, dinv, conv_w[N_CLUSTERS:]).reshape(N_CLUSTERS * NP, HID)
    segp2 = seg_kernel(ys2, fused, col3, zeros128)
    (pooled2,) = _layer(segp2, ys2.reshape(N_CLUSTERS, NP, HID), dinv,
                        lab_pad, batch_pad, emit_x2=False)

    return _head(pooled1, pooled2, mlp_w1, mlp_b1, bn_gamma, bn_beta,
                 mlp_w2, mlp_b2)


# speedup vs baseline: 1.1527x; 1.1527x over previous
import functools

import jax
import jax.numpy as jnp
from jax import lax
from jax.experimental import pallas as pl
from jax.experimental.pallas import tpu as pltpu
from jax.experimental.pallas import tpu_sc as plsc

N_CLUSTERS = 4
HID = 128
N_GRAPHS = 8

NP = 10240
NW = 32
CHUNK = 128
NCHUNK = 79
EP = NW * NCHUNK * CHUNK
ROWS_PER_TILE = NP // 16


def _deg_body(col_hbm, ones_hbm, zeros_hbm, out_hbm, colbuf, onesbuf, acc):
    cid = lax.axis_index("c")
    sid = lax.axis_index("s")
    w = cid * 16 + sid
    base = sid * ROWS_PER_TILE
    pltpu.sync_copy(zeros_hbm.at[pl.ds(base, ROWS_PER_TILE)],
                    acc.at[pl.ds(base, ROWS_PER_TILE)])
    pltpu.sync_copy(col_hbm.at[w], colbuf)
    pltpu.sync_copy(ones_hbm, onesbuf)
    plsc.subcore_barrier()

    def body(j, carry):
        pltpu.sync_copy(onesbuf, acc.at[colbuf.at[j]], add=True)
        return carry

    lax.fori_loop(0, NCHUNK, body, 0)
    plsc.subcore_barrier()
    pltpu.sync_copy(acc.at[pl.ds(base, ROWS_PER_TILE)],
                    out_hbm.at[cid, pl.ds(base, ROWS_PER_TILE)])


def _seg_body(tab_hbm, idx_hbm, col_hbm, zeros_hbm, out_hbm,
              idxbuf, colbuf, rows_v, acc, sem):
    cid = lax.axis_index("c")
    sid = lax.axis_index("s")
    w = cid * 16 + sid
    base = sid * ROWS_PER_TILE
    pltpu.sync_copy(zeros_hbm.at[pl.ds(base, ROWS_PER_TILE)],
                    acc.at[pl.ds(base, ROWS_PER_TILE)])
    pltpu.sync_copy(idx_hbm.at[w], idxbuf)
    pltpu.sync_copy(col_hbm.at[w], colbuf)
    plsc.subcore_barrier()

    def body(j, carry):
        pltpu.async_copy(tab_hbm.at[idxbuf.at[j]], rows_v, sem).wait()
        pltpu.sync_copy(rows_v, acc.at[colbuf.at[j]], add=True)
        return carry

    lax.fori_loop(0, NCHUNK, body, 0)
    plsc.subcore_barrier()
    pltpu.sync_copy(acc.at[pl.ds(base, ROWS_PER_TILE)],
                    out_hbm.at[cid, pl.ds(base, ROWS_PER_TILE)])


@functools.lru_cache(maxsize=None)
def _sc_kernels():
    mesh = plsc.VectorSubcoreMesh(core_axis_name="c", subcore_axis_name="s")
    deg = pl.kernel(
        _deg_body,
        out_type=jax.ShapeDtypeStruct((2, NP, HID), jnp.float32),
        mesh=mesh,
        scratch_types=[
            pltpu.VMEM((NCHUNK, CHUNK), jnp.int32),
            pltpu.VMEM((CHUNK, HID), jnp.float32),
            pltpu.VMEM_SHARED((NP, HID), jnp.float32),
        ],
    )
    seg = pl.kernel(
        _seg_body,
        out_type=jax.ShapeDtypeStruct((2, NP, HID), jnp.float32),
        mesh=mesh,
        scratch_types=[
            pltpu.VMEM((NCHUNK, CHUNK), jnp.int32),
            pltpu.VMEM((NCHUNK, CHUNK), jnp.int32),
            pltpu.VMEM((CHUNK, HID), jnp.float32),
            pltpu.VMEM_SHARED((NP, HID), jnp.float32),
            pltpu.SemaphoreType.DMA,
        ],
    )
    return deg, seg


_BLK = 1024


def _dinv_body(degp_ref, dinv_ref):
    deg = degp_ref[0, :, 0] + degp_ref[1, :, 0] + 1.0
    dinv_ref[...] = lax.rsqrt(deg)[:, None]


def _dinv(degp):
    grid = NP // _BLK
    return pl.pallas_call(
        _dinv_body,
        grid=(grid,),
        in_specs=[pl.BlockSpec((2, _BLK, HID), lambda i: (0, i, 0))],
        out_specs=pl.BlockSpec((_BLK, 1), lambda i: (i, 0)),
        out_shape=jax.ShapeDtypeStruct((NP, 1), jnp.float32),
    )(degp)


def _ys_body(h_ref, dinv_ref, w_ref, ys_ref):
    h = h_ref[...]
    dinv = dinv_ref[...]
    for c in range(N_CLUSTERS):
        ys_ref[c] = dinv * jnp.dot(h, w_ref[c],
                                   preferred_element_type=jnp.float32)


def _ys(h, dinv, w4):
    grid = NP // _BLK
    return pl.pallas_call(
        _ys_body,
        grid=(grid,),
        in_specs=[
            pl.BlockSpec((_BLK, HID), lambda i: (i, 0)),
            pl.BlockSpec((_BLK, 1), lambda i: (i, 0)),
            pl.BlockSpec((N_CLUSTERS, HID, HID), lambda i: (0, 0, 0)),
        ],
        out_specs=pl.BlockSpec((N_CLUSTERS, _BLK, HID), lambda i: (0, i, 0)),
        out_shape=jax.ShapeDtypeStruct((N_CLUSTERS, NP, HID), jnp.float32),
    )(h, dinv, w4)


def _layer_body(emit_x2, seg_ref, ys_ref, dinv_ref, lab_ref, bat_ref,
                *out_refs):
    i = pl.program_id(0)
    lab = lab_ref[...]
    ys_sel = jnp.zeros((_BLK, HID), jnp.float32)
    for c in range(N_CLUSTERS):
        ys_sel = jnp.where(lab == c, ys_ref[c], ys_sel)
    x2 = dinv_ref[...] * (seg_ref[0] + seg_ref[1] + ys_sel)
    x2 = jnp.maximum(x2, 0.0)
    if emit_x2:
        out_refs[0][...] = x2
    pooled_ref = out_refs[-1]
    onehot = (bat_ref[...] == lax.broadcasted_iota(jnp.int32, (1, N_GRAPHS), 1)
              ).astype(jnp.float32)
    p = lax.dot_general(onehot, x2, (((0,), (0,)), ((), ())),
                        preferred_element_type=jnp.float32,
                        precision=lax.Precision.HIGHEST)

    @pl.when(i == 0)
    def _():
        pooled_ref[...] = jnp.zeros_like(pooled_ref)

    pooled_ref[...] += p


def _layer(segp, ys, dinv, lab, batch, emit_x2):
    grid = NP // _BLK
    out_shape = [jax.ShapeDtypeStruct((N_GRAPHS, HID), jnp.float32)]
    out_specs = [pl.BlockSpec((N_GRAPHS, HID), lambda i: (0, 0))]
    if emit_x2:
        out_shape.insert(0, jax.ShapeDtypeStruct((NP, HID), jnp.float32))
        out_specs.insert(0, pl.BlockSpec((_BLK, HID), lambda i: (i, 0)))
    return pl.pallas_call(
        functools.partial(_layer_body, emit_x2),
        grid=(grid,),
        in_specs=[
            pl.BlockSpec((2, _BLK, HID), lambda i: (0, i, 0)),
            pl.BlockSpec((N_CLUSTERS, _BLK, HID), lambda i: (0, i, 0)),
            pl.BlockSpec((_BLK, 1), lambda i: (i, 0)),
            pl.BlockSpec((_BLK, 1), lambda i: (i, 0)),
            pl.BlockSpec((_BLK, 1), lambda i: (i, 0)),
        ],
        out_specs=out_specs,
        out_shape=out_shape,
    )(segp, ys, dinv, lab, batch)


def _head_body(p1_ref, p2_ref, w1_ref, b1_ref, g_ref, b_ref, w2_ref, b2_ref,
               out_ref):
    z = jnp.concatenate([p1_ref[...], p2_ref[...]], axis=1)
    z1 = jnp.dot(z, w1_ref[...], preferred_element_type=jnp.float32) + b1_ref[...]
    mean = jnp.mean(z1, axis=0, keepdims=True)
    var = jnp.mean(jnp.square(z1 - mean), axis=0, keepdims=True)
    zn = (z1 - mean) * lax.rsqrt(var + 1e-5) * g_ref[...] + b_ref[...]
    zn = jnp.maximum(zn, 0.0)
    out_ref[...] = jnp.dot(zn, w2_ref[...],
                           preferred_element_type=jnp.float32) + b2_ref[...]


def _head(p1, p2, w1, b1, gamma, beta, w2, b2):
    return pl.pallas_call(
        _head_body,
        out_shape=jax.ShapeDtypeStruct((N_GRAPHS, 64), jnp.float32),
    )(p1, p2, w1, b1.reshape(1, -1), gamma.reshape(1, -1),
      beta.reshape(1, -1), w2, b2.reshape(1, -1))


def kernel(x, edge_index, batch, conv_w, mlp_w1, mlp_b1, bn_gamma, bn_beta,
           mlp_w2, mlp_b2):
    lab = x[:, :1].astype(jnp.int32)
    h = x[:, 1:]
    n, _ = h.shape

    h_pad = jnp.pad(h, ((0, NP - n), (0, 0)))
    lab_pad = jnp.pad(lab, ((0, NP - n), (0, 0)))
    batch_pad = jnp.pad(batch.astype(jnp.int32), (0, NP - n)).reshape(NP, 1)

    row = edge_index[0].astype(jnp.int32)
    col = edge_index[1].astype(jnp.int32)
    e = row.shape[0]
    rowp = jnp.pad(row, (0, EP - e), constant_values=n)
    colp = jnp.pad(col, (0, EP - e), constant_values=n)
    lab_col = lab_pad[colp, 0]
    fused = (lab_col * NP + rowp).reshape(NW, NCHUNK, CHUNK)
    col3 = colp.reshape(NW, NCHUNK, CHUNK)

    ones128 = jnp.ones((CHUNK, HID), jnp.float32)
    zeros128 = jnp.zeros((NP, HID), jnp.float32)

    deg_kernel, seg_kernel = _sc_kernels()
    degp = deg_kernel(col3, ones128, zeros128)
    dinv = _dinv(degp)

    ys1 = _ys(h_pad, dinv, conv_w[:N_CLUSTERS]).reshape(N_CLUSTERS * NP, HID)
    segp1 = seg_kernel(ys1, fused, col3, zeros128)
    x2, pooled1 = _layer(segp1, ys1.reshape(N_CLUSTERS, NP, HID), dinv,
                         lab_pad, batch_pad, emit_x2=True)

    ys2 = _ys(x2, dinv, conv_w[N_CLUSTERS:]).reshape(N_CLUSTERS * NP, HID)
    segp2 = seg_kernel(ys2, fused, col3, zeros128)
    (pooled2,) = _layer(segp2, ys2.reshape(N_CLUSTERS, NP, HID), dinv,
                        lab_pad, batch_pad, emit_x2=False)

    return _head(pooled1, pooled2, mlp_w1, mlp_b1, bn_gamma, bn_beta,
                 mlp_w2, mlp_b2)
